# baseline (speedup 1.0000x reference)
import functools

import jax
import jax.numpy as jnp
from jax import lax
from jax.experimental import pallas as pl
from jax.experimental.pallas import tpu as pltpu
from jax.experimental.pallas import tpu_sc as plsc

BATCH = 16384
D = 64
L = 16
NC, NS = 2, 16
NW = NC * NS
BPW = BATCH // NW
NQ = 4
QB = BPW // NQ

_mesh = plsc.VectorSubcoreMesh(core_axis_name="c", subcore_axis_name="s")


@functools.partial(
    pl.kernel,
    out_type=(
        jax.ShapeDtypeStruct((BATCH,), jnp.float32),
        jax.ShapeDtypeStruct((BATCH,), jnp.float32),
    ),
    mesh=_mesh,
    compiler_params=pltpu.CompilerParams(
        needs_layout_passes=False, use_tc_tiling_on_sc=False),
    scratch_types=[
        pltpu.VMEM((NQ, QB), jnp.int32),
        pltpu.VMEM((NQ, QB), jnp.int32),
        pltpu.VMEM((NQ, QB), jnp.int32),
        pltpu.VMEM((NQ, QB, D), jnp.float32),
        pltpu.VMEM((NQ, QB, D), jnp.float32),
        pltpu.VMEM((NQ, QB, D), jnp.float32),
        pltpu.VMEM((BPW,), jnp.float32),
        pltpu.VMEM((BPW,), jnp.float32),
        pltpu.SemaphoreType.DMA,
    ],
)
def _sc_kernel(uids_hbm, pids_hbm, nids_hbm, utab_hbm, itab_hbm,
               pos_hbm, neg_hbm,
               uidx, pidx, nidx, urows, prows, nrows, posv, negv, sem):
    wid = lax.axis_index("s") * NC + lax.axis_index("c")
    base = wid * BPW

    for q in range(NQ):
        pltpu.sync_copy(uids_hbm.at[pl.ds(base + q * QB, QB)], uidx.at[q])
        pltpu.sync_copy(pids_hbm.at[pl.ds(base + q * QB, QB)], pidx.at[q])
        pltpu.sync_copy(nids_hbm.at[pl.ds(base + q * QB, QB)], nidx.at[q])

    copies = []
    for q in range(NQ):
        copies.append(pltpu.async_copy(utab_hbm.at[uidx.at[q]], urows.at[q], sem))
        copies.append(pltpu.async_copy(itab_hbm.at[pidx.at[q]], prows.at[q], sem))
        copies.append(pltpu.async_copy(itab_hbm.at[nidx.at[q]], nrows.at[q], sem))
    for c in copies:
        c.wait()

    lanes = lax.iota(jnp.int32, L)

    def chunk(c, carry):
        q = c // (QB // L)
        r0 = (c % (QB // L)) * L
        pvec = jnp.zeros((L,), jnp.float32)
        nvec = jnp.zeros((L,), jnp.float32)
        for j in range(L):
            r = r0 + j
            tp = jnp.zeros((L,), jnp.float32)
            tn = jnp.zeros((L,), jnp.float32)
            for k in range(D // L):
                u = urows[q, r, pl.ds(k * L, L)]
                tp = tp + u * prows[q, r, pl.ds(k * L, L)]
                tn = tn + u * nrows[q, r, pl.ds(k * L, L)]
            sp = jnp.sum(tp)
            sn = jnp.sum(tn)
            pvec = jnp.where(lanes == j, sp, pvec)
            nvec = jnp.where(lanes == j, sn, nvec)
        posv[pl.ds(c * L, L)] = pvec
        negv[pl.ds(c * L, L)] = nvec
        return carry

    lax.fori_loop(0, BPW // L, chunk, 0)

    pltpu.sync_copy(posv, pos_hbm.at[pl.ds(base, BPW)])
    pltpu.sync_copy(negv, neg_hbm.at[pl.ds(base, BPW)])


def kernel(user_ids, pos_ids, neg_ids, user_table, item_table):
    return _sc_kernel(user_ids, pos_ids, neg_ids, user_table, item_table)
